# baseline (speedup 1.0000x reference)
import functools

import jax
import jax.numpy as jnp
from jax import lax
from jax.experimental import pallas as pl
from jax.experimental.pallas import tpu as pltpu
from jax.experimental.pallas import tpu_sc as plsc

N = 10000
NP = 10112
E = 320000
EP = 327680
G = 64
H = 128
NC, NS = 2, 16
CH = 64
RPT = (EP // CH) // NS
NBLK = RPT // 8
CH2 = 128
RPT2 = (EP // CH2) // NS
NBLK2 = RPT2 // 4
HALF = NP // NC
TRASH = HALF
AROWS = 5120
ASTRIPE = AROWS // NS

_mesh = plsc.VectorSubcoreMesh(core_axis_name="c", subcore_axis_name="s")


def _remap_dst(didx, c, nrows, ncol):
    lo = c * HALF
    for r in range(nrows):
        for j in range(ncol // 16):
            v = didx[r, pl.ds(j * 16, 16)]
            inside = jnp.logical_and(v >= lo, v < lo + HALF)
            didx[r, pl.ds(j * 16, 16)] = jnp.where(inside, v - lo, TRASH)


def _zero_fill(buf, nrows, ncol):
    def zrow(i, carry):
        for j in range(ncol // 16):
            buf[i, pl.ds(j * 16, 16)] = jnp.zeros((16,), jnp.float32)
        return carry
    lax.fori_loop(0, nrows, zrow, 0)


def _zero_fill1(buf, nwords16):
    def zrow(i, carry):
        buf[pl.ds(i * 16, 16)] = jnp.zeros((16,), jnp.float32)
        return carry
    lax.fori_loop(0, nwords16, zrow, 0)


def _zero_acc2(rows, acc, s):
    _zero_fill(rows, CH, H)
    base = s * ASTRIPE
    for q in range(5):
        pltpu.sync_copy(rows.at[pl.ds(0, 64)], acc.at[pl.ds(base + q * 64, 64)])


def _zero_acc1(stg, acc, s):
    _zero_fill1(stg, 8)
    base = s * ASTRIPE
    pltpu.sync_copy(stg.at[pl.ds(0, 128)], acc.at[pl.ds(base, 128)])
    pltpu.sync_copy(stg.at[pl.ds(0, 128)], acc.at[pl.ds(base + 128, 128)])
    pltpu.sync_copy(stg.at[pl.ds(0, 64)], acc.at[pl.ds(base + 256, 64)])


def _writeout2(rows, acc, out, c, s):
    lbase = s * 320
    gbase = c * HALF + lbase

    def chunk(off):
        pltpu.sync_copy(acc.at[pl.ds(lbase + off, 64)], rows.at[pl.ds(0, 64)])
        pltpu.sync_copy(rows.at[pl.ds(0, 64)], out.at[pl.ds(gbase + off, 64)])

    @pl.when(s < 15)
    def _():
        for q in range(5):
            chunk(q * 64)

    @pl.when(s == 15)
    def _():
        for q in range(4):
            chunk(q * 64)


def _writeout1(stg, acc, out, c, s):
    lbase = s * 320
    gbase = c * HALF + lbase

    def chunk(sz, off):
        pltpu.sync_copy(acc.at[pl.ds(lbase + off, sz)], stg.at[pl.ds(0, sz)])
        pltpu.sync_copy(stg.at[pl.ds(0, sz)], out.at[pl.ds(gbase + off, sz)])

    @pl.when(s < 15)
    def _():
        chunk(128, 0)
        chunk(128, 128)
        chunk(64, 256)

    @pl.when(s == 15)
    def _():
        chunk(128, 0)
        chunk(128, 128)


def _sc_indeg_body(didxr, out, didx, ones, stg, acc, sems):
    c = lax.axis_index("c")
    s = lax.axis_index("s")
    _zero_acc1(stg, acc, s)
    for r in range(4):
        for j in range(CH2 // 16):
            ones[r, pl.ds(j * 16, 16)] = jnp.ones((16,), jnp.float32)
    plsc.subcore_barrier()

    def blk(b, carry):
        pltpu.sync_copy(didxr.at[s].at[pl.ds(b * 4, 4)], didx)
        _remap_dst(didx, c, 4, CH2)
        sds = [pltpu.async_copy(ones.at[r], acc.at[didx.at[r]], sems, add=True)
               for r in range(4)]
        for d in sds:
            d.wait()
        return carry

    lax.fori_loop(0, NBLK2, blk, 0)
    plsc.subcore_barrier()
    _writeout1(stg, acc, out, c, s)


def _make_sc_indeg():
    return functools.partial(
        pl.kernel,
        out_type=jax.ShapeDtypeStruct((NP,), jnp.float32),
        mesh=_mesh,
        scratch_types=[
            pltpu.VMEM((4, CH2), jnp.int32),
            pltpu.VMEM((4, CH2), jnp.float32),
            pltpu.VMEM((128,), jnp.float32),
            pltpu.VMEM_SHARED((AROWS,), jnp.float32),
            pltpu.SemaphoreType.DMA,
        ],
    )(_sc_indeg_body)


def _sc_row_agg_body(hp, sidxr, didxr, out, sidx, didx, rows0, rows1,
                     acc, semg, sems):
    c = lax.axis_index("c")
    s = lax.axis_index("s")
    _zero_acc2(rows0, acc, s)
    plsc.subcore_barrier()
    bufs = (rows0, rows1)

    def blk(b, carry):
        pltpu.sync_copy(sidxr.at[s].at[pl.ds(b * 8, 8)], sidx)
        pltpu.sync_copy(didxr.at[s].at[pl.ds(b * 8, 8)], didx)
        _remap_dst(didx, c, 8, CH)
        gd = [None] * 8
        sd = [None] * 8
        gd[0] = pltpu.async_copy(hp.at[sidx.at[0]], bufs[0], semg)
        for r in range(8):
            gd[r].wait()
            sd[r] = pltpu.async_copy(bufs[r % 2], acc.at[didx.at[r]], sems,
                                     add=True)
            if r < 7:
                if r >= 1:
                    sd[r - 1].wait()
                gd[r + 1] = pltpu.async_copy(hp.at[sidx.at[r + 1]],
                                             bufs[(r + 1) % 2], semg)
        sd[6].wait()
        sd[7].wait()
        return carry

    lax.fori_loop(0, NBLK, blk, 0)
    plsc.subcore_barrier()
    _writeout2(rows0, acc, out, c, s)


def _make_sc_row_agg():
    return functools.partial(
        pl.kernel,
        out_type=jax.ShapeDtypeStruct((NP, H), jnp.float32),
        mesh=_mesh,
        scratch_types=[
            pltpu.VMEM((8, CH), jnp.int32),
            pltpu.VMEM((8, CH), jnp.int32),
            pltpu.VMEM((CH, H), jnp.float32),
            pltpu.VMEM((CH, H), jnp.float32),
            pltpu.VMEM_SHARED((AROWS, H), jnp.float32),
            pltpu.SemaphoreType.DMA,
            pltpu.SemaphoreType.DMA,
        ],
    )(_sc_row_agg_body)


def _sc_scal_agg_body(v, sidxr, didxr, out, sidx, didx, vals, stg, acc, semg):
    c = lax.axis_index("c")
    s = lax.axis_index("s")
    _zero_acc1(stg, acc, s)
    plsc.subcore_barrier()

    def blk(b, carry):
        pltpu.sync_copy(sidxr.at[s].at[pl.ds(b * 4, 4)], sidx)
        pltpu.sync_copy(didxr.at[s].at[pl.ds(b * 4, 4)], didx)
        _remap_dst(didx, c, 4, CH2)
        gds = [pltpu.async_copy(v.at[sidx.at[r]], vals.at[r], semg)
               for r in range(4)]
        for d in gds:
            d.wait()
        sds = [pltpu.async_copy(vals.at[r], acc.at[didx.at[r]], semg, add=True)
               for r in range(4)]
        for d in sds:
            d.wait()
        return carry

    lax.fori_loop(0, NBLK2, blk, 0)
    plsc.subcore_barrier()
    _writeout1(stg, acc, out, c, s)


def _make_sc_scal_agg():
    return functools.partial(
        pl.kernel,
        out_type=jax.ShapeDtypeStruct((NP,), jnp.float32),
        mesh=_mesh,
        scratch_types=[
            pltpu.VMEM((4, CH2), jnp.int32),
            pltpu.VMEM((4, CH2), jnp.int32),
            pltpu.VMEM((4, CH2), jnp.float32),
            pltpu.VMEM((128,), jnp.float32),
            pltpu.VMEM_SHARED((AROWS,), jnp.float32),
            pltpu.SemaphoreType.DMA,
        ],
    )(_sc_scal_agg_body)


_BLK = 1264


def _tcb_body(x_ref, w_ref, i0_ref, m_ref, hp_ref, dis_ref):
    deg = i0_ref[...] + m_ref[...]
    pos = deg > 0
    dis = jnp.where(pos, lax.rsqrt(jnp.where(pos, deg, 1.0)), 0.0)
    h = jnp.dot(x_ref[...], w_ref[...], preferred_element_type=jnp.float32)
    hp_ref[...] = h * dis
    dis_ref[...] = dis


def _tc_b(xp, W1, i0, m0):
    grid = NP // _BLK
    return pl.pallas_call(
        _tcb_body,
        grid=(grid,),
        in_specs=[
            pl.BlockSpec((_BLK, H), lambda k: (k, 0)),
            pl.BlockSpec((H, H), lambda k: (0, 0)),
            pl.BlockSpec((_BLK, 1), lambda k: (k, 0)),
            pl.BlockSpec((_BLK, 1), lambda k: (k, 0)),
        ],
        out_specs=[
            pl.BlockSpec((_BLK, H), lambda k: (k, 0)),
            pl.BlockSpec((_BLK, 1), lambda k: (k, 0)),
        ],
        out_shape=[
            jax.ShapeDtypeStruct((NP, H), jnp.float32),
            jax.ShapeDtypeStruct((NP, 1), jnp.float32),
        ],
    )(xp, W1, i0, m0)


def _tcd_body(a0, hp, dis, m, b1, wp, bp, x1o, hs0o, t0o):
    d = dis[...]
    mm = m[...]
    agg = d * a0[...] + d * hp[...] * mm
    x1 = jax.nn.relu((agg + b1[...]) * mm)
    x1o[...] = x1
    hs0 = jnp.dot(x1, wp[...], preferred_element_type=jnp.float32) * d
    hs0o[...] = hs0
    t0o[...] = d * hs0 + bp[...]


def _tc_d(a0, hp1, dis1, m0, b1, Wp0, bp0):
    grid = NP // _BLK
    return pl.pallas_call(
        _tcd_body,
        grid=(grid,),
        in_specs=[
            pl.BlockSpec((_BLK, H), lambda k: (k, 0)),
            pl.BlockSpec((_BLK, H), lambda k: (k, 0)),
            pl.BlockSpec((_BLK, 1), lambda k: (k, 0)),
            pl.BlockSpec((_BLK, 1), lambda k: (k, 0)),
            pl.BlockSpec((1, H), lambda k: (0, 0)),
            pl.BlockSpec((H, 1), lambda k: (0, 0)),
            pl.BlockSpec((1, 1), lambda k: (0, 0)),
        ],
        out_specs=[
            pl.BlockSpec((_BLK, H), lambda k: (k, 0)),
            pl.BlockSpec((_BLK, 1), lambda k: (k, 0)),
            pl.BlockSpec((_BLK, 1), lambda k: (k, 0)),
        ],
        out_shape=[
            jax.ShapeDtypeStruct((NP, H), jnp.float32),
            jax.ShapeDtypeStruct((NP, 1), jnp.float32),
            jax.ShapeDtypeStruct((NP, 1), jnp.float32),
        ],
    )(a0, hp1, dis1, m0, b1, Wp0, bp0)


def _tcf_body(x1, s0, m1, p0, w2, hp2o, dis2o):
    m = m1[...]
    dis2 = m * lax.rsqrt(p0[...] + 1.0)
    x1p = x1[...] * jnp.tanh(s0[...]) * m
    hp2o[...] = jnp.dot(x1p, w2[...], preferred_element_type=jnp.float32) * dis2
    dis2o[...] = dis2


def _tc_f(x1, s0c, m1c, p0, W2):
    grid = NP // _BLK
    return pl.pallas_call(
        _tcf_body,
        grid=(grid,),
        in_specs=[
            pl.BlockSpec((_BLK, H), lambda k: (k, 0)),
            pl.BlockSpec((_BLK, 1), lambda k: (k, 0)),
            pl.BlockSpec((_BLK, 1), lambda k: (k, 0)),
            pl.BlockSpec((_BLK, 1), lambda k: (k, 0)),
            pl.BlockSpec((H, H), lambda k: (0, 0)),
        ],
        out_specs=[
            pl.BlockSpec((_BLK, H), lambda k: (k, 0)),
            pl.BlockSpec((_BLK, 1), lambda k: (k, 0)),
        ],
        out_shape=[
            jax.ShapeDtypeStruct((NP, H), jnp.float32),
            jax.ShapeDtypeStruct((NP, 1), jnp.float32),
        ],
    )(x1, s0c, m1c, p0, W2)


def _tch_body(c0, hp2, dis2, m1, b2, wp1, bp1, x2o, hs1o, t1o):
    d = dis2[...]
    m = m1[...]
    agg = d * c0[...] + d * hp2[...] * m
    x2 = jax.nn.relu((agg + b2[...]) * m)
    x2o[...] = x2
    hs1 = jnp.dot(x2, wp1[...], preferred_element_type=jnp.float32) * d
    hs1o[...] = hs1
    t1o[...] = d * hs1 + bp1[...]


def _tc_h(c0, hp2, dis2, m1c, b2, Wp1, bp1):
    grid = NP // _BLK
    return pl.pallas_call(
        _tch_body,
        grid=(grid,),
        in_specs=[
            pl.BlockSpec((_BLK, H), lambda k: (k, 0)),
            pl.BlockSpec((_BLK, H), lambda k: (k, 0)),
            pl.BlockSpec((_BLK, 1), lambda k: (k, 0)),
            pl.BlockSpec((_BLK, 1), lambda k: (k, 0)),
            pl.BlockSpec((1, H), lambda k: (0, 0)),
            pl.BlockSpec((H, 1), lambda k: (0, 0)),
            pl.BlockSpec((1, 1), lambda k: (0, 0)),
        ],
        out_specs=[
            pl.BlockSpec((_BLK, H), lambda k: (k, 0)),
            pl.BlockSpec((_BLK, 1), lambda k: (k, 0)),
            pl.BlockSpec((_BLK, 1), lambda k: (k, 0)),
        ],
        out_shape=[
            jax.ShapeDtypeStruct((NP, H), jnp.float32),
            jax.ShapeDtypeStruct((NP, 1), jnp.float32),
            jax.ShapeDtypeStruct((NP, 1), jnp.float32),
        ],
    )(c0, hp2, dis2, m1c, b2, Wp1, bp1)


NPOOL = 10240
_PB = 1280
_PC = 1280
_PNJ = NPOOL // _PC


def _pool_body(p0, t, dis, mask, batch, s_o, m_o, rank_acc, s_scr):
    k = pl.program_id(0)
    mfull = mask[...]
    sfull = mfull * (dis[...] * p0[...] + t[...])
    s_scr[...] = sfull
    bfull = batch[...]
    gio = lax.broadcasted_iota(jnp.int32, (G, 1), 0)
    onehot = (bfull == gio).astype(jnp.float32)
    kept = jnp.sum(onehot * mfull, axis=1, keepdims=True)
    kcap = jnp.ceil(0.5 * kept)
    s_blk = s_scr[:, pl.ds(k * _PB, _PB)]
    s_i = jnp.swapaxes(s_blk, 0, 1)
    s_o[...] = s_i
    b_blk = batch[:, pl.ds(k * _PB, _PB)]
    b_i = jnp.swapaxes(b_blk, 0, 1)
    m_blk = mask[:, pl.ds(k * _PB, _PB)]
    m_i = jnp.swapaxes(m_blk, 0, 1)
    idx_i = k * _PB + lax.broadcasted_iota(jnp.int32, (_PB, 1), 0)
    g_lo = jnp.min(b_i)
    g_hi = jnp.max(b_i)
    rank_acc[...] = jnp.zeros((_PB, 1), jnp.float32)
    for jc in range(_PNJ):
        jb = batch[0, jc * _PC]
        je = batch[0, jc * _PC + _PC - 1]

        @pl.when(jnp.logical_and(jb <= g_hi, je >= g_lo))
        def _():
            s_j = s_scr[:, pl.ds(jc * _PC, _PC)]
            b_j = batch[:, pl.ds(jc * _PC, _PC)]
            m_j = mask[:, pl.ds(jc * _PC, _PC)]
            idx_j = jc * _PC + lax.broadcasted_iota(jnp.int32, (1, _PC), 1)
            cmp = (s_j > s_i) | ((s_j == s_i) & (idx_j < idx_i))
            ok = cmp & (b_j == b_i) & (m_j > 0)
            rank_acc[...] += jnp.sum(ok.astype(jnp.float32), axis=1, keepdims=True)

    rank = rank_acc[...]
    oh_i = (b_i == lax.broadcasted_iota(jnp.int32, (1, G), 1)).astype(jnp.float32)
    kcap_i = jnp.dot(oh_i, kcap, preferred_element_type=jnp.float32)
    m_new = m_i * (rank < kcap_i).astype(jnp.float32)
    m_o[...] = m_new


def _tc_pool(p0r, tr, disr, maskr, batchr):
    grid = NPOOL // _PB
    full_f = pl.BlockSpec((1, NPOOL), lambda k: (0, 0))
    return pl.pallas_call(
        _pool_body,
        grid=(grid,),
        in_specs=[full_f, full_f, full_f, full_f,
                  pl.BlockSpec((1, NPOOL), lambda k: (0, 0))],
        out_specs=[
            pl.BlockSpec((_PB, 1), lambda k: (k, 0)),
            pl.BlockSpec((_PB, 1), lambda k: (k, 0)),
        ],
        out_shape=[
            jax.ShapeDtypeStruct((NPOOL, 1), jnp.float32),
            jax.ShapeDtypeStruct((NPOOL, 1), jnp.float32),
        ],
        scratch_shapes=[pltpu.VMEM((_PB, 1), jnp.float32),
                        pltpu.VMEM((1, NPOOL), jnp.float32)],
    )(p0r, tr, disr, maskr, batchr)


def _ro_body(x2, s1, m2, bt, l1w, l1b, l2w, l2b, out, accf, accc):
    k = pl.program_id(0)

    @pl.when(k == 0)
    def _():
        accf[...] = jnp.zeros((G, H), jnp.float32)
        accc[...] = jnp.zeros((G, 1), jnp.float32)

    s1b = s1[...]
    m2b = m2[...]
    bt_row = jnp.swapaxes(bt[...], 0, 1)
    scale = jnp.tanh(s1b) * m2b
    xt = x2[...] * scale
    oh = (bt_row == lax.broadcasted_iota(jnp.int32, (G, 1), 0)).astype(jnp.float32)
    accf[...] += jnp.dot(oh, xt, preferred_element_type=jnp.float32)
    accc[...] += jnp.dot(oh, m2b, preferred_element_type=jnp.float32)

    @pl.when(k == NP // _BLK - 1)
    def _():
        gm = accf[...] / jnp.maximum(accc[...], 1.0)
        h = jax.nn.relu(jnp.dot(gm, l1w[...], preferred_element_type=jnp.float32)
                        + l1b[...])
        logits = jnp.dot(h, l2w[...], preferred_element_type=jnp.float32) + l2b[...]
        mx = jnp.max(logits, axis=-1, keepdims=True)
        sh = logits - mx
        out[...] = sh - jnp.log(jnp.sum(jnp.exp(sh), axis=-1, keepdims=True))


def _tc_readout(x2, s1r, m2r, batchr, L1W, L1b, L2W, L2b):
    grid = NP // _BLK
    return pl.pallas_call(
        _ro_body,
        grid=(grid,),
        in_specs=[
            pl.BlockSpec((_BLK, H), lambda k: (k, 0)),
            pl.BlockSpec((_BLK, 1), lambda k: (k, 0)),
            pl.BlockSpec((_BLK, 1), lambda k: (k, 0)),
            pl.BlockSpec((_BLK, 1), lambda k: (k, 0)),
            pl.BlockSpec((H, H), lambda k: (0, 0)),
            pl.BlockSpec((1, H), lambda k: (0, 0)),
            pl.BlockSpec((H, 10), lambda k: (0, 0)),
            pl.BlockSpec((1, 10), lambda k: (0, 0)),
        ],
        out_specs=pl.BlockSpec((G, 10), lambda k: (0, 0)),
        out_shape=jax.ShapeDtypeStruct((G, 10), jnp.float32),
        scratch_shapes=[
            pltpu.VMEM((G, H), jnp.float32),
            pltpu.VMEM((G, 1), jnp.float32),
        ],
    )(x2, s1r, m2r, batchr, L1W, L1b, L2W, L2b)


def kernel(x, edge_index, batch, W1, b1, Wp0, bp0, W2, b2, Wp1, bp1, L1W, L1b, L2W, L2b):
    f32 = jnp.float32
    src = edge_index[0].astype(jnp.int32)
    dst = edge_index[1].astype(jnp.int32)
    srcp = jnp.concatenate([src, jnp.zeros((EP - E,), jnp.int32)])
    dstp = jnp.concatenate([dst, jnp.full((EP - E,), NP - 1, jnp.int32)])
    sidxr = srcp.reshape(NS, RPT, CH)
    didxr = dstp.reshape(NS, RPT, CH)
    sidxr2 = srcp.reshape(NS, RPT2, CH2)
    didxr2 = dstp.reshape(NS, RPT2, CH2)
    batch = batch.astype(jnp.int32)

    xp = jnp.pad(x, ((0, NP - N), (0, 0)))
    m0col = jnp.pad(jnp.ones((N, 1), f32), ((0, NP - N), (0, 0)))
    b1r = b1.reshape(1, H)
    b2r = b2.reshape(1, H)
    bp0r = bp0.reshape(1, 1)
    bp1r = bp1.reshape(1, 1)
    batch_pool = jnp.pad(batch, (0, NPOOL - N), constant_values=G).reshape(1, NPOOL)
    batch_col = jnp.pad(batch, (0, NP - N), constant_values=G).reshape(NP, 1)

    def prow(a):
        return jnp.pad(a.reshape(-1)[:N], (0, NPOOL - N)).reshape(1, NPOOL)

    indeg = _make_sc_indeg()(didxr2)
    i0 = indeg.reshape(NP, 1)

    hp1, dis1 = _tc_b(xp, W1, i0, m0col)
    acc1 = _make_sc_row_agg()(hp1, sidxr, didxr)
    x1, hs0, t0 = _tc_d(acc1, hp1, dis1, m0col, b1r, Wp0, bp0r)

    accs0 = _make_sc_scal_agg()(hs0.reshape(NP), sidxr2, didxr2)
    mask0_row = prow(jnp.ones((N,), f32))
    s0p_, m1p_ = _tc_pool(prow(accs0), prow(t0), prow(dis1), mask0_row, batch_pool)

    m1p = jnp.pad(m1p_.reshape(-1)[:N], (0, NP - N))
    p2 = _make_sc_scal_agg()(m1p, sidxr2, didxr2)
    s0c = jnp.pad(s0p_.reshape(-1)[:N], (0, NP - N)).reshape(NP, 1)
    m1c = m1p.reshape(NP, 1)
    hp2, dis2 = _tc_f(x1, s0c, m1c, p2.reshape(NP, 1), W2)
    acc2 = _make_sc_row_agg()(hp2, sidxr, didxr)
    x2, hs1, t1 = _tc_h(acc2, hp2, dis2, m1c, b2r, Wp1, bp1r)

    accs1 = _make_sc_scal_agg()(hs1.reshape(NP), sidxr2, didxr2)
    s1p_, m2p_ = _tc_pool(prow(accs1), prow(t1), prow(dis2), prow(m1p), batch_pool)

    s1pc = jnp.pad(s1p_.reshape(-1)[:N], (0, NP - N)).reshape(NP, 1)
    m2pc = jnp.pad(m2p_.reshape(-1)[:N], (0, NP - N)).reshape(NP, 1)
    return _tc_readout(x2, s1pc, m2pc, batch_col, L1W, L1b.reshape(1, H),
                       L2W, L2b.reshape(1, 10))


# speedup vs baseline: 1.1367x; 1.1367x over previous
import functools

import jax
import jax.numpy as jnp
from jax import lax
from jax.experimental import pallas as pl
from jax.experimental.pallas import tpu as pltpu
from jax.experimental.pallas import tpu_sc as plsc

N = 10000
NP = 10112
E = 320000
EP = 327680
G = 64
H = 128
NC, NS = 2, 16
CH = 64
RPT = (EP // CH) // NS
NBLK = RPT // 8
CH2 = 128
RPT2 = (EP // CH2) // NS
NBLK2 = RPT2 // 4
HALF = NP // NC
TRASH = HALF
AROWS = 5120
ASTRIPE = AROWS // NS

_mesh = plsc.VectorSubcoreMesh(core_axis_name="c", subcore_axis_name="s")


def _remap_dst(didx, c, nrows, ncol):
    lo = c * HALF
    for r in range(nrows):
        for j in range(ncol // 16):
            v = didx[r, pl.ds(j * 16, 16)]
            inside = jnp.logical_and(v >= lo, v < lo + HALF)
            trash = TRASH + (v & 63)
            didx[r, pl.ds(j * 16, 16)] = jnp.where(inside, v - lo, trash)


def _zero_fill(buf, nrows, ncol):
    def zrow(i, carry):
        for j in range(ncol // 16):
            buf[i, pl.ds(j * 16, 16)] = jnp.zeros((16,), jnp.float32)
        return carry
    lax.fori_loop(0, nrows, zrow, 0)


def _zero_fill1(buf, nwords16):
    def zrow(i, carry):
        buf[pl.ds(i * 16, 16)] = jnp.zeros((16,), jnp.float32)
        return carry
    lax.fori_loop(0, nwords16, zrow, 0)


def _zero_acc2(rows, acc, s):
    _zero_fill(rows, CH, H)
    base = s * ASTRIPE
    for q in range(5):
        pltpu.sync_copy(rows.at[pl.ds(0, 64)], acc.at[pl.ds(base + q * 64, 64)])


def _zero_acc1(stg, acc, s):
    _zero_fill1(stg, 8)
    base = s * ASTRIPE
    pltpu.sync_copy(stg.at[pl.ds(0, 128)], acc.at[pl.ds(base, 128)])
    pltpu.sync_copy(stg.at[pl.ds(0, 128)], acc.at[pl.ds(base + 128, 128)])
    pltpu.sync_copy(stg.at[pl.ds(0, 64)], acc.at[pl.ds(base + 256, 64)])


def _writeout2(rows, acc, out, c, s):
    lbase = s * 320
    gbase = c * HALF + lbase

    def chunk(off):
        pltpu.sync_copy(acc.at[pl.ds(lbase + off, 64)], rows.at[pl.ds(0, 64)])
        pltpu.sync_copy(rows.at[pl.ds(0, 64)], out.at[pl.ds(gbase + off, 64)])

    @pl.when(s < 15)
    def _():
        for q in range(5):
            chunk(q * 64)

    @pl.when(s == 15)
    def _():
        for q in range(4):
            chunk(q * 64)


def _writeout1(stg, acc, out, c, s):
    lbase = s * 320
    gbase = c * HALF + lbase

    def chunk(sz, off):
        pltpu.sync_copy(acc.at[pl.ds(lbase + off, sz)], stg.at[pl.ds(0, sz)])
        pltpu.sync_copy(stg.at[pl.ds(0, sz)], out.at[pl.ds(gbase + off, sz)])

    @pl.when(s < 15)
    def _():
        chunk(128, 0)
        chunk(128, 128)
        chunk(64, 256)

    @pl.when(s == 15)
    def _():
        chunk(128, 0)
        chunk(128, 128)


def _sc_indeg_body(didxr, out, didx, ones, stg, acc, sems):
    c = lax.axis_index("c")
    s = lax.axis_index("s")
    _zero_acc1(stg, acc, s)
    for r in range(4):
        for j in range(CH2 // 16):
            ones[r, pl.ds(j * 16, 16)] = jnp.ones((16,), jnp.float32)
    plsc.subcore_barrier()

    def blk(b, carry):
        pltpu.sync_copy(didxr.at[s].at[pl.ds(b * 4, 4)], didx)
        _remap_dst(didx, c, 4, CH2)
        sds = [pltpu.async_copy(ones.at[r], acc.at[didx.at[r]], sems, add=True)
               for r in range(4)]
        for d in sds:
            d.wait()
        return carry

    lax.fori_loop(0, NBLK2, blk, 0)
    plsc.subcore_barrier()
    _writeout1(stg, acc, out, c, s)


def _make_sc_indeg():
    return functools.partial(
        pl.kernel,
        out_type=jax.ShapeDtypeStruct((NP,), jnp.float32),
        mesh=_mesh,
        scratch_types=[
            pltpu.VMEM((4, CH2), jnp.int32),
            pltpu.VMEM((4, CH2), jnp.float32),
            pltpu.VMEM((128,), jnp.float32),
            pltpu.VMEM_SHARED((AROWS,), jnp.float32),
            pltpu.SemaphoreType.DMA,
        ],
    )(_sc_indeg_body)


def _sc_row_agg_body(hp, sidxr, didxr, out, sidx, didx, rows0, rows1,
                     acc, semg, sems):
    c = lax.axis_index("c")
    s = lax.axis_index("s")
    _zero_acc2(rows0, acc, s)
    plsc.subcore_barrier()
    bufs = (rows0, rows1)

    def blk(b, carry):
        pltpu.sync_copy(sidxr.at[s].at[pl.ds(b * 8, 8)], sidx)
        pltpu.sync_copy(didxr.at[s].at[pl.ds(b * 8, 8)], didx)
        _remap_dst(didx, c, 8, CH)
        gd = [None] * 8
        sd = [None] * 8
        gd[0] = pltpu.async_copy(hp.at[sidx.at[0]], bufs[0], semg)
        for r in range(8):
            gd[r].wait()
            sd[r] = pltpu.async_copy(bufs[r % 2], acc.at[didx.at[r]], sems,
                                     add=True)
            if r < 7:
                if r >= 1:
                    sd[r - 1].wait()
                gd[r + 1] = pltpu.async_copy(hp.at[sidx.at[r + 1]],
                                             bufs[(r + 1) % 2], semg)
        sd[6].wait()
        sd[7].wait()
        return carry

    lax.fori_loop(0, NBLK, blk, 0)
    plsc.subcore_barrier()
    _writeout2(rows0, acc, out, c, s)


def _make_sc_row_agg():
    return functools.partial(
        pl.kernel,
        out_type=jax.ShapeDtypeStruct((NP, H), jnp.float32),
        mesh=_mesh,
        scratch_types=[
            pltpu.VMEM((8, CH), jnp.int32),
            pltpu.VMEM((8, CH), jnp.int32),
            pltpu.VMEM((CH, H), jnp.float32),
            pltpu.VMEM((CH, H), jnp.float32),
            pltpu.VMEM_SHARED((AROWS, H), jnp.float32),
            pltpu.SemaphoreType.DMA,
            pltpu.SemaphoreType.DMA,
        ],
    )(_sc_row_agg_body)


def _sc_scal_agg_body(v, sidxr, didxr, out, sidx, didx, vals, stg, acc, semg):
    c = lax.axis_index("c")
    s = lax.axis_index("s")
    _zero_acc1(stg, acc, s)
    plsc.subcore_barrier()

    def blk(b, carry):
        pltpu.sync_copy(sidxr.at[s].at[pl.ds(b * 4, 4)], sidx)
        pltpu.sync_copy(didxr.at[s].at[pl.ds(b * 4, 4)], didx)
        _remap_dst(didx, c, 4, CH2)
        gds = [pltpu.async_copy(v.at[sidx.at[r]], vals.at[r], semg)
               for r in range(4)]
        for d in gds:
            d.wait()
        sds = [pltpu.async_copy(vals.at[r], acc.at[didx.at[r]], semg, add=True)
               for r in range(4)]
        for d in sds:
            d.wait()
        return carry

    lax.fori_loop(0, NBLK2, blk, 0)
    plsc.subcore_barrier()
    _writeout1(stg, acc, out, c, s)


def _make_sc_scal_agg():
    return functools.partial(
        pl.kernel,
        out_type=jax.ShapeDtypeStruct((NP,), jnp.float32),
        mesh=_mesh,
        scratch_types=[
            pltpu.VMEM((4, CH2), jnp.int32),
            pltpu.VMEM((4, CH2), jnp.int32),
            pltpu.VMEM((4, CH2), jnp.float32),
            pltpu.VMEM((128,), jnp.float32),
            pltpu.VMEM_SHARED((AROWS,), jnp.float32),
            pltpu.SemaphoreType.DMA,
        ],
    )(_sc_scal_agg_body)


_BLK = 1264


def _tcb_body(x_ref, w_ref, i0_ref, m_ref, hp_ref, dis_ref):
    deg = i0_ref[...] + m_ref[...]
    pos = deg > 0
    dis = jnp.where(pos, lax.rsqrt(jnp.where(pos, deg, 1.0)), 0.0)
    h = jnp.dot(x_ref[...], w_ref[...], preferred_element_type=jnp.float32)
    hp_ref[...] = h * dis
    dis_ref[...] = dis


def _tc_b(xp, W1, i0, m0):
    grid = NP // _BLK
    return pl.pallas_call(
        _tcb_body,
        grid=(grid,),
        in_specs=[
            pl.BlockSpec((_BLK, H), lambda k: (k, 0)),
            pl.BlockSpec((H, H), lambda k: (0, 0)),
            pl.BlockSpec((_BLK, 1), lambda k: (k, 0)),
            pl.BlockSpec((_BLK, 1), lambda k: (k, 0)),
        ],
        out_specs=[
            pl.BlockSpec((_BLK, H), lambda k: (k, 0)),
            pl.BlockSpec((_BLK, 1), lambda k: (k, 0)),
        ],
        out_shape=[
            jax.ShapeDtypeStruct((NP, H), jnp.float32),
            jax.ShapeDtypeStruct((NP, 1), jnp.float32),
        ],
    )(xp, W1, i0, m0)


def _tcd_body(a0, hp, dis, m, b1, wp, bp, x1o, hs0o, t0o):
    d = dis[...]
    mm = m[...]
    agg = d * a0[...] + d * hp[...] * mm
    x1 = jax.nn.relu((agg + b1[...]) * mm)
    x1o[...] = x1
    hs0 = jnp.dot(x1, wp[...], preferred_element_type=jnp.float32) * d
    hs0o[...] = hs0
    t0o[...] = d * hs0 + bp[...]


def _tc_d(a0, hp1, dis1, m0, b1, Wp0, bp0):
    grid = NP // _BLK
    return pl.pallas_call(
        _tcd_body,
        grid=(grid,),
        in_specs=[
            pl.BlockSpec((_BLK, H), lambda k: (k, 0)),
            pl.BlockSpec((_BLK, H), lambda k: (k, 0)),
            pl.BlockSpec((_BLK, 1), lambda k: (k, 0)),
            pl.BlockSpec((_BLK, 1), lambda k: (k, 0)),
            pl.BlockSpec((1, H), lambda k: (0, 0)),
            pl.BlockSpec((H, 1), lambda k: (0, 0)),
            pl.BlockSpec((1, 1), lambda k: (0, 0)),
        ],
        out_specs=[
            pl.BlockSpec((_BLK, H), lambda k: (k, 0)),
            pl.BlockSpec((_BLK, 1), lambda k: (k, 0)),
            pl.BlockSpec((_BLK, 1), lambda k: (k, 0)),
        ],
        out_shape=[
            jax.ShapeDtypeStruct((NP, H), jnp.float32),
            jax.ShapeDtypeStruct((NP, 1), jnp.float32),
            jax.ShapeDtypeStruct((NP, 1), jnp.float32),
        ],
    )(a0, hp1, dis1, m0, b1, Wp0, bp0)


def _tcf_body(x1, s0, m1, p0, w2, hp2o, dis2o):
    m = m1[...]
    dis2 = m * lax.rsqrt(p0[...] + 1.0)
    x1p = x1[...] * jnp.tanh(s0[...]) * m
    hp2o[...] = jnp.dot(x1p, w2[...], preferred_element_type=jnp.float32) * dis2
    dis2o[...] = dis2


def _tc_f(x1, s0c, m1c, p0, W2):
    grid = NP // _BLK
    return pl.pallas_call(
        _tcf_body,
        grid=(grid,),
        in_specs=[
            pl.BlockSpec((_BLK, H), lambda k: (k, 0)),
            pl.BlockSpec((_BLK, 1), lambda k: (k, 0)),
            pl.BlockSpec((_BLK, 1), lambda k: (k, 0)),
            pl.BlockSpec((_BLK, 1), lambda k: (k, 0)),
            pl.BlockSpec((H, H), lambda k: (0, 0)),
        ],
        out_specs=[
            pl.BlockSpec((_BLK, H), lambda k: (k, 0)),
            pl.BlockSpec((_BLK, 1), lambda k: (k, 0)),
        ],
        out_shape=[
            jax.ShapeDtypeStruct((NP, H), jnp.float32),
            jax.ShapeDtypeStruct((NP, 1), jnp.float32),
        ],
    )(x1, s0c, m1c, p0, W2)


def _tch_body(c0, hp2, dis2, m1, b2, wp1, bp1, x2o, hs1o, t1o):
    d = dis2[...]
    m = m1[...]
    agg = d * c0[...] + d * hp2[...] * m
    x2 = jax.nn.relu((agg + b2[...]) * m)
    x2o[...] = x2
    hs1 = jnp.dot(x2, wp1[...], preferred_element_type=jnp.float32) * d
    hs1o[...] = hs1
    t1o[...] = d * hs1 + bp1[...]


def _tc_h(c0, hp2, dis2, m1c, b2, Wp1, bp1):
    grid = NP // _BLK
    return pl.pallas_call(
        _tch_body,
        grid=(grid,),
        in_specs=[
            pl.BlockSpec((_BLK, H), lambda k: (k, 0)),
            pl.BlockSpec((_BLK, H), lambda k: (k, 0)),
            pl.BlockSpec((_BLK, 1), lambda k: (k, 0)),
            pl.BlockSpec((_BLK, 1), lambda k: (k, 0)),
            pl.BlockSpec((1, H), lambda k: (0, 0)),
            pl.BlockSpec((H, 1), lambda k: (0, 0)),
            pl.BlockSpec((1, 1), lambda k: (0, 0)),
        ],
        out_specs=[
            pl.BlockSpec((_BLK, H), lambda k: (k, 0)),
            pl.BlockSpec((_BLK, 1), lambda k: (k, 0)),
            pl.BlockSpec((_BLK, 1), lambda k: (k, 0)),
        ],
        out_shape=[
            jax.ShapeDtypeStruct((NP, H), jnp.float32),
            jax.ShapeDtypeStruct((NP, 1), jnp.float32),
            jax.ShapeDtypeStruct((NP, 1), jnp.float32),
        ],
    )(c0, hp2, dis2, m1c, b2, Wp1, bp1)


NPOOL = 10240
_PB = 1280
_PC = 1280
_PNJ = NPOOL // _PC


def _pool_body(p0, t, dis, mask, batch, s_o, m_o, rank_acc, s_scr):
    k = pl.program_id(0)
    mfull = mask[...]
    sfull = mfull * (dis[...] * p0[...] + t[...])
    s_scr[...] = sfull
    bfull = batch[...]
    gio = lax.broadcasted_iota(jnp.int32, (G, 1), 0)
    onehot = (bfull == gio).astype(jnp.float32)
    kept = jnp.sum(onehot * mfull, axis=1, keepdims=True)
    kcap = jnp.ceil(0.5 * kept)
    s_blk = s_scr[:, pl.ds(k * _PB, _PB)]
    s_i = jnp.swapaxes(s_blk, 0, 1)
    s_o[...] = s_i
    b_blk = batch[:, pl.ds(k * _PB, _PB)]
    b_i = jnp.swapaxes(b_blk, 0, 1)
    m_blk = mask[:, pl.ds(k * _PB, _PB)]
    m_i = jnp.swapaxes(m_blk, 0, 1)
    idx_i = k * _PB + lax.broadcasted_iota(jnp.int32, (_PB, 1), 0)
    g_lo = jnp.min(b_i)
    g_hi = jnp.max(b_i)
    rank_acc[...] = jnp.zeros((_PB, 1), jnp.float32)
    for jc in range(_PNJ):
        jb = batch[0, jc * _PC]
        je = batch[0, jc * _PC + _PC - 1]

        @pl.when(jnp.logical_and(jb <= g_hi, je >= g_lo))
        def _():
            s_j = s_scr[:, pl.ds(jc * _PC, _PC)]
            b_j = batch[:, pl.ds(jc * _PC, _PC)]
            m_j = mask[:, pl.ds(jc * _PC, _PC)]
            idx_j = jc * _PC + lax.broadcasted_iota(jnp.int32, (1, _PC), 1)
            cmp = (s_j > s_i) | ((s_j == s_i) & (idx_j < idx_i))
            ok = cmp & (b_j == b_i) & (m_j > 0)
            rank_acc[...] += jnp.sum(ok.astype(jnp.float32), axis=1, keepdims=True)

    rank = rank_acc[...]
    oh_i = (b_i == lax.broadcasted_iota(jnp.int32, (1, G), 1)).astype(jnp.float32)
    kcap_i = jnp.dot(oh_i, kcap, preferred_element_type=jnp.float32)
    m_new = m_i * (rank < kcap_i).astype(jnp.float32)
    m_o[...] = m_new


def _tc_pool(p0r, tr, disr, maskr, batchr):
    grid = NPOOL // _PB
    full_f = pl.BlockSpec((1, NPOOL), lambda k: (0, 0))
    return pl.pallas_call(
        _pool_body,
        grid=(grid,),
        in_specs=[full_f, full_f, full_f, full_f,
                  pl.BlockSpec((1, NPOOL), lambda k: (0, 0))],
        out_specs=[
            pl.BlockSpec((_PB, 1), lambda k: (k, 0)),
            pl.BlockSpec((_PB, 1), lambda k: (k, 0)),
        ],
        out_shape=[
            jax.ShapeDtypeStruct((NPOOL, 1), jnp.float32),
            jax.ShapeDtypeStruct((NPOOL, 1), jnp.float32),
        ],
        scratch_shapes=[pltpu.VMEM((_PB, 1), jnp.float32),
                        pltpu.VMEM((1, NPOOL), jnp.float32)],
    )(p0r, tr, disr, maskr, batchr)


def _ro_body(x2, s1, m2, bt, l1w, l1b, l2w, l2b, out, accf, accc):
    k = pl.program_id(0)

    @pl.when(k == 0)
    def _():
        accf[...] = jnp.zeros((G, H), jnp.float32)
        accc[...] = jnp.zeros((G, 1), jnp.float32)

    s1b = s1[...]
    m2b = m2[...]
    bt_row = jnp.swapaxes(bt[...], 0, 1)
    scale = jnp.tanh(s1b) * m2b
    xt = x2[...] * scale
    oh = (bt_row == lax.broadcasted_iota(jnp.int32, (G, 1), 0)).astype(jnp.float32)
    accf[...] += jnp.dot(oh, xt, preferred_element_type=jnp.float32)
    accc[...] += jnp.dot(oh, m2b, preferred_element_type=jnp.float32)

    @pl.when(k == NP // _BLK - 1)
    def _():
        gm = accf[...] / jnp.maximum(accc[...], 1.0)
        h = jax.nn.relu(jnp.dot(gm, l1w[...], preferred_element_type=jnp.float32)
                        + l1b[...])
        logits = jnp.dot(h, l2w[...], preferred_element_type=jnp.float32) + l2b[...]
        mx = jnp.max(logits, axis=-1, keepdims=True)
        sh = logits - mx
        out[...] = sh - jnp.log(jnp.sum(jnp.exp(sh), axis=-1, keepdims=True))


def _tc_readout(x2, s1r, m2r, batchr, L1W, L1b, L2W, L2b):
    grid = NP // _BLK
    return pl.pallas_call(
        _ro_body,
        grid=(grid,),
        in_specs=[
            pl.BlockSpec((_BLK, H), lambda k: (k, 0)),
            pl.BlockSpec((_BLK, 1), lambda k: (k, 0)),
            pl.BlockSpec((_BLK, 1), lambda k: (k, 0)),
            pl.BlockSpec((_BLK, 1), lambda k: (k, 0)),
            pl.BlockSpec((H, H), lambda k: (0, 0)),
            pl.BlockSpec((1, H), lambda k: (0, 0)),
            pl.BlockSpec((H, 10), lambda k: (0, 0)),
            pl.BlockSpec((1, 10), lambda k: (0, 0)),
        ],
        out_specs=pl.BlockSpec((G, 10), lambda k: (0, 0)),
        out_shape=jax.ShapeDtypeStruct((G, 10), jnp.float32),
        scratch_shapes=[
            pltpu.VMEM((G, H), jnp.float32),
            pltpu.VMEM((G, 1), jnp.float32),
        ],
    )(x2, s1r, m2r, batchr, L1W, L1b, L2W, L2b)


def kernel(x, edge_index, batch, W1, b1, Wp0, bp0, W2, b2, Wp1, bp1, L1W, L1b, L2W, L2b):
    f32 = jnp.float32
    src = edge_index[0].astype(jnp.int32)
    dst = edge_index[1].astype(jnp.int32)
    srcp = jnp.concatenate([src, jnp.zeros((EP - E,), jnp.int32)])
    dstp = jnp.concatenate([dst, jnp.full((EP - E,), NP - 1, jnp.int32)])
    sidxr = srcp.reshape(NS, RPT, CH)
    didxr = dstp.reshape(NS, RPT, CH)
    sidxr2 = srcp.reshape(NS, RPT2, CH2)
    didxr2 = dstp.reshape(NS, RPT2, CH2)
    batch = batch.astype(jnp.int32)

    xp = jnp.pad(x, ((0, NP - N), (0, 0)))
    m0col = jnp.pad(jnp.ones((N, 1), f32), ((0, NP - N), (0, 0)))
    b1r = b1.reshape(1, H)
    b2r = b2.reshape(1, H)
    bp0r = bp0.reshape(1, 1)
    bp1r = bp1.reshape(1, 1)
    batch_pool = jnp.pad(batch, (0, NPOOL - N), constant_values=G).reshape(1, NPOOL)
    batch_col = jnp.pad(batch, (0, NP - N), constant_values=G).reshape(NP, 1)

    def prow(a):
        return jnp.pad(a.reshape(-1)[:N], (0, NPOOL - N)).reshape(1, NPOOL)

    indeg = _make_sc_indeg()(didxr2)
    i0 = indeg.reshape(NP, 1)

    hp1, dis1 = _tc_b(xp, W1, i0, m0col)
    acc1 = _make_sc_row_agg()(hp1, sidxr, didxr)
    x1, hs0, t0 = _tc_d(acc1, hp1, dis1, m0col, b1r, Wp0, bp0r)

    accs0 = _make_sc_scal_agg()(hs0.reshape(NP), sidxr2, didxr2)
    mask0_row = prow(jnp.ones((N,), f32))
    s0p_, m1p_ = _tc_pool(prow(accs0), prow(t0), prow(dis1), mask0_row, batch_pool)

    m1p = jnp.pad(m1p_.reshape(-1)[:N], (0, NP - N))
    p2 = _make_sc_scal_agg()(m1p, sidxr2, didxr2)
    s0c = jnp.pad(s0p_.reshape(-1)[:N], (0, NP - N)).reshape(NP, 1)
    m1c = m1p.reshape(NP, 1)
    hp2, dis2 = _tc_f(x1, s0c, m1c, p2.reshape(NP, 1), W2)
    acc2 = _make_sc_row_agg()(hp2, sidxr, didxr)
    x2, hs1, t1 = _tc_h(acc2, hp2, dis2, m1c, b2r, Wp1, bp1r)

    accs1 = _make_sc_scal_agg()(hs1.reshape(NP), sidxr2, didxr2)
    s1p_, m2p_ = _tc_pool(prow(accs1), prow(t1), prow(dis2), prow(m1p), batch_pool)

    s1pc = jnp.pad(s1p_.reshape(-1)[:N], (0, NP - N)).reshape(NP, 1)
    m2pc = jnp.pad(m2p_.reshape(-1)[:N], (0, NP - N)).reshape(NP, 1)
    return _tc_readout(x2, s1pc, m2pc, batch_col, L1W, L1b.reshape(1, H),
                       L2W, L2b.reshape(1, 10))
